# Initial kernel scaffold; baseline (speedup 1.0000x reference)
import jax
import jax.numpy as jnp
from jax.experimental import pallas as pl


def kernel(x, edge_index, W1, b1, W2, b2, W3, b3):
    raise NotImplementedError("write your pallas kernel here")


# speedup vs baseline: 17.5498x; 17.5498x over previous
import functools

import jax
import jax.numpy as jnp
from jax import lax
from jax.experimental import pallas as pl
from jax.experimental.pallas import tpu as pltpu
from jax.experimental.pallas import tpu_sc as plsc

N_NODES = 10000
D = 128
E = 320000

NC = 2
NS = 16
NW = NC * NS
NP = 10240
RPS = NP // NS
EPW = E // NW
K = 100
NCH = EPW // K

_mesh = plsc.VectorSubcoreMesh(
    core_axis_name="c", subcore_axis_name="s", num_cores=NC, num_subcores=NS
)


@functools.partial(
    pl.kernel,
    out_type=jax.ShapeDtypeStruct((NC, NP), jnp.float32),
    mesh=_mesh,
    scratch_types=[
        pltpu.VMEM((NCH, K), jnp.int32),
        pltpu.VMEM((K,), jnp.float32),
        pltpu.VMEM_SHARED((NP,), jnp.float32),
    ],
)
def _sc_degree(dst_hbm, ones_hbm, zeros_hbm, out_hbm, dst_v, ones_v, deg_sh):
    cid = lax.axis_index("c")
    sid = lax.axis_index("s")
    wid = sid * NC + cid
    pltpu.sync_copy(zeros_hbm, deg_sh.at[pl.ds(sid * RPS, RPS)])
    pltpu.sync_copy(dst_hbm.at[wid], dst_v)
    pltpu.sync_copy(ones_hbm, ones_v)
    plsc.subcore_barrier()

    def body(j, carry):
        pltpu.sync_copy(ones_v, deg_sh.at[dst_v.at[j]], add=True)
        return carry

    lax.fori_loop(0, NCH, body, 0)
    plsc.subcore_barrier()
    pltpu.sync_copy(
        deg_sh.at[pl.ds(sid * RPS, RPS)], out_hbm.at[cid, pl.ds(sid * RPS, RPS)]
    )


@functools.partial(
    pl.kernel,
    out_type=jax.ShapeDtypeStruct((NC, NP, D), jnp.float32),
    mesh=_mesh,
    scratch_types=[
        pltpu.VMEM((NCH, K), jnp.int32),
        pltpu.VMEM((NCH, K), jnp.int32),
        pltpu.VMEM((K, D), jnp.float32),
        pltpu.VMEM_SHARED((NP, D), jnp.float32),
        pltpu.SemaphoreType.DMA,
    ],
)
def _sc_aggregate(
    h_hbm, src_hbm, dst_hbm, zeros_hbm, out_hbm, src_v, dst_v, rows_v, acc_sh, sem
):
    cid = lax.axis_index("c")
    sid = lax.axis_index("s")
    wid = sid * NC + cid
    pltpu.sync_copy(zeros_hbm, acc_sh.at[pl.ds(sid * RPS, RPS)])
    pltpu.sync_copy(src_hbm.at[wid], src_v)
    pltpu.sync_copy(dst_hbm.at[wid], dst_v)
    plsc.subcore_barrier()

    def body(j, carry):
        pltpu.async_copy(h_hbm.at[src_v.at[j]], rows_v, sem).wait()
        pltpu.sync_copy(rows_v, acc_sh.at[dst_v.at[j]], add=True)
        return carry

    lax.fori_loop(0, NCH, body, 0)
    plsc.subcore_barrier()
    pltpu.sync_copy(
        acc_sh.at[pl.ds(sid * RPS, RPS)], out_hbm.at[cid, pl.ds(sid * RPS, RPS)]
    )


def _dense_first_body(deg_ref, x_ref, w_ref, h2_ref, dinv_ref):
    deg = deg_ref[:, 0:1] + deg_ref[:, 1:2] + 1.0
    dinv = lax.rsqrt(deg)
    h = jnp.dot(x_ref[...], w_ref[...], preferred_element_type=jnp.float32)
    h2_ref[...] = h * dinv
    dinv_ref[...] = dinv


def _dense_mid_body(acc_ref, h2_ref, dinv_ref, b_ref, w_ref, out_ref):
    tot = acc_ref[0] + acc_ref[1] + h2_ref[...]
    y = jnp.maximum(tot * dinv_ref[...] + b_ref[...], 0.0)
    out_ref[...] = (
        jnp.dot(y, w_ref[...], preferred_element_type=jnp.float32) * dinv_ref[...]
    )


def _dense_last_body(acc_ref, h2_ref, dinv_ref, b_ref, out_ref):
    tot = acc_ref[0] + acc_ref[1] + h2_ref[...]
    out_ref[...] = jnp.maximum(tot * dinv_ref[...] + b_ref[...], 0.0)


_f32 = jnp.float32

_dense_first = pl.pallas_call(
    _dense_first_body,
    out_shape=(
        jax.ShapeDtypeStruct((NP, D), _f32),
        jax.ShapeDtypeStruct((NP, 1), _f32),
    ),
)

_dense_mid = pl.pallas_call(
    _dense_mid_body,
    out_shape=jax.ShapeDtypeStruct((NP, D), _f32),
)

_dense_last = pl.pallas_call(
    _dense_last_body,
    out_shape=jax.ShapeDtypeStruct((NP, D), _f32),
)


def kernel(x, edge_index, W1, b1, W2, b2, W3, b3):
    src = edge_index[0].astype(jnp.int32).reshape(NW, NCH, K)
    dst = edge_index[1].astype(jnp.int32).reshape(NW, NCH, K)
    xp = jnp.pad(x, ((0, NP - N_NODES), (0, 0)))
    ones_k = jnp.ones((K,), _f32)
    zeros_1d = jnp.zeros((RPS,), _f32)
    zeros_2d = jnp.zeros((RPS, D), _f32)

    deg2 = _sc_degree(dst, ones_k, zeros_1d)
    h2, dinv = _dense_first(deg2.T, xp, W1)
    acc = _sc_aggregate(h2, src, dst, zeros_2d)
    h2 = _dense_mid(acc, h2, dinv, b1.reshape(1, D), W2)
    acc = _sc_aggregate(h2, src, dst, zeros_2d)
    h2 = _dense_mid(acc, h2, dinv, b2.reshape(1, D), W3)
    acc = _sc_aggregate(h2, src, dst, zeros_2d)
    y = _dense_last(acc, h2, dinv, b3.reshape(1, D))
    return y[:N_NODES]
